# Initial kernel scaffold; baseline (speedup 1.0000x reference)
import jax
import jax.numpy as jnp
from jax.experimental import pallas as pl


def kernel(nfeat, edge_index, W_neigh, b_neigh):
    raise NotImplementedError("write your pallas kernel here")


# speedup vs baseline: 7.3044x; 7.3044x over previous
import functools

import jax
import jax.numpy as jnp
from jax import lax
from jax.experimental import pallas as pl
from jax.experimental.pallas import tpu as pltpu
from jax.experimental.pallas import tpu_sc as plsc

N_NODES = 10000
N_PAD = 10240
N_EDGES = 320000
CHUNK = 128
N_CHUNKS = N_EDGES // CHUNK
NC, NS = 2, 16
NW = NC * NS
ROWS_PER_TILE = N_PAD // NS
D = 128
FULL_CNT = N_CHUNKS // NW
EXTRA = N_CHUNKS - NW * FULL_CNT


def _sc_scatter(nfeat, src2d, dst2d):
    mesh = plsc.VectorSubcoreMesh(core_axis_name="c", subcore_axis_name="s")

    @functools.partial(
        pl.kernel,
        out_type=(
            jax.ShapeDtypeStruct((NC, N_PAD, D), jnp.float32),
            jax.ShapeDtypeStruct((NC, N_PAD), jnp.float32),
        ),
        mesh=mesh,
        scratch_types=[
            pltpu.VMEM_SHARED((N_PAD, D), jnp.float32),
            pltpu.VMEM_SHARED((N_PAD,), jnp.float32),
            pltpu.VMEM((CHUNK,), jnp.int32),
            pltpu.VMEM((CHUNK,), jnp.int32),
            pltpu.VMEM((CHUNK, D), jnp.float32),
            pltpu.VMEM((CHUNK,), jnp.float32),
            pltpu.VMEM((64, D), jnp.float32),
            pltpu.VMEM((ROWS_PER_TILE,), jnp.float32),
            pltpu.SemaphoreType.DMA,
        ],
    )
    def k(nfeat_h, src_h, dst_h, out_h, deg_h,
          acc_sh, deg_sh, src_v, dst_v, rows_v, ones_v, zrow_v, z1d_v, sem):
        c = lax.axis_index("c")
        s = lax.axis_index("s")
        wid = s * NC + c

        zeros16 = jnp.zeros((16,), jnp.float32)
        ones16 = jnp.ones((16,), jnp.float32)

        def zrow_body(i, carry):
            for j in range(D // 16):
                zrow_v[i, pl.ds(j * 16, 16)] = zeros16
            return carry

        lax.fori_loop(0, 64, zrow_body, 0)

        def z1d_body(i, carry):
            z1d_v[pl.ds(i * 16, 16)] = zeros16
            return carry

        lax.fori_loop(0, ROWS_PER_TILE // 16, z1d_body, 0)

        for j in range(CHUNK // 16):
            ones_v[pl.ds(j * 16, 16)] = ones16

        row0 = s * ROWS_PER_TILE
        for t in range(ROWS_PER_TILE // 64):
            pltpu.sync_copy(zrow_v, acc_sh.at[pl.ds(row0 + t * 64, 64)])
        pltpu.sync_copy(z1d_v, deg_sh.at[pl.ds(row0, ROWS_PER_TILE)])
        plsc.subcore_barrier()

        count = FULL_CNT + jnp.where(wid < EXTRA, 1, 0)

        def body(t, carry):
            j = wid + NW * t
            pltpu.sync_copy(src_h.at[j], src_v)
            pltpu.sync_copy(dst_h.at[j], dst_v)
            pltpu.async_copy(nfeat_h.at[src_v], rows_v, sem).wait()
            pltpu.sync_copy(rows_v, acc_sh.at[dst_v], add=True)
            pltpu.sync_copy(ones_v, deg_sh.at[dst_v], add=True)
            return carry

        lax.fori_loop(0, count, body, 0)
        plsc.subcore_barrier()

        pltpu.sync_copy(acc_sh.at[pl.ds(row0, ROWS_PER_TILE)],
                        out_h.at[c, pl.ds(row0, ROWS_PER_TILE)])
        pltpu.sync_copy(deg_sh.at[pl.ds(row0, ROWS_PER_TILE)],
                        deg_h.at[c, pl.ds(row0, ROWS_PER_TILE)])

    return k(nfeat, src2d, dst2d)


def _tc_combine(nf_pad, p0, p1, d0, d1, W, b2d):
    BLK = 1024
    grid = (N_PAD // BLK,)

    def body(nf, p0r, p1r, d0r, d1r, wr, br, o):
        d = d0r[...] + d1r[...] + 1.0
        r = 1.0 / d
        sfeat = p0r[...] + p1r[...] + 2.0 * nf[...]
        y = lax.dot_general(sfeat, wr[...], (((1,), (1,)), ((), ())),
                            preferred_element_type=jnp.float32)
        o[...] = y * r + br[...] * (1.0 + r)

    return pl.pallas_call(
        body,
        grid=grid,
        in_specs=[
            pl.BlockSpec((BLK, D), lambda i: (i, 0)),
            pl.BlockSpec((BLK, D), lambda i: (i, 0)),
            pl.BlockSpec((BLK, D), lambda i: (i, 0)),
            pl.BlockSpec((BLK, 1), lambda i: (i, 0)),
            pl.BlockSpec((BLK, 1), lambda i: (i, 0)),
            pl.BlockSpec((D, D), lambda i: (0, 0)),
            pl.BlockSpec((1, D), lambda i: (0, 0)),
        ],
        out_specs=pl.BlockSpec((BLK, D), lambda i: (i, 0)),
        out_shape=jax.ShapeDtypeStruct((N_PAD, D), jnp.float32),
    )(nf_pad, p0, p1, d0, d1, W, b2d)


def kernel(nfeat, edge_index, W_neigh, b_neigh):
    src2d = edge_index[0].astype(jnp.int32).reshape(N_CHUNKS, CHUNK)
    dst2d = edge_index[1].astype(jnp.int32).reshape(N_CHUNKS, CHUNK)
    partial, deg = _sc_scatter(nfeat, src2d, dst2d)
    nf_pad = jnp.zeros((N_PAD, D), jnp.float32).at[:N_NODES].set(nfeat)
    out = _tc_combine(nf_pad, partial[0], partial[1],
                      deg[0].reshape(N_PAD, 1), deg[1].reshape(N_PAD, 1),
                      W_neigh, b_neigh.reshape(1, D))
    return out[:N_NODES]
